# Initial kernel scaffold; baseline (speedup 1.0000x reference)
import jax
import jax.numpy as jnp
from jax.experimental import pallas as pl


def kernel(x, edge_index, edge_attr, W1, att_src1, att_dst1, We1, att_e1, b1, W2, att_src2, att_dst2, We2, att_e2, b2):
    raise NotImplementedError("write your pallas kernel here")


# speedup vs baseline: 7.9173x; 7.9173x over previous
import functools

import jax
import jax.numpy as jnp
from jax import lax
from jax.experimental import pallas as pl
from jax.experimental.pallas import tpu as pltpu
from jax.experimental.pallas import tpu_sc as plsc

N = 10000
E = 320000
IN_DIM = 128
HID = 256
HALF = 128
NC = 2
NS = 16
L = 16

E_PAD = 327680
CHUNK_B = E_PAD // (NC * NS)
NBLK_B = CHUNK_B // L
CHUNK_C = E_PAD // NS
NBLK_C = CHUNK_C // 128
N_PAD = 10240
ROWS_T = N_PAD // NS
ROW_CH = 128

_mesh = plsc.VectorSubcoreMesh(core_axis_name="c", subcore_axis_name="s")


def _dense_body(x_ref, w_ref, ha0, ha1, ha2, ha3, aug_ref, *, relu):
    ha_ref = (ha0, ha1, ha2, ha3)
    xb = x_ref[...]
    if relu:
        xb = jnp.maximum(xb, 0.0)
    h = jnp.dot(xb, w_ref[...], preferred_element_type=jnp.float32)
    for q in range(4):
        ha_ref[q][...] = h[:, q * 64:(q + 1) * 64]
    aug_ref[...] = h[:, HID:]


def _dense(x, w_aug, relu):
    rows = 1000
    grid = (N // rows,)
    in_dim = x.shape[1]
    return pl.pallas_call(
        functools.partial(_dense_body, relu=relu),
        grid=grid,
        in_specs=[
            pl.BlockSpec((rows, in_dim), lambda i: (i, 0)),
            pl.BlockSpec((in_dim, HID + 128), lambda i: (0, 0)),
        ],
        out_specs=[pl.BlockSpec((rows, 64), lambda i: (i, 0))] * 4 + [
            pl.BlockSpec((rows, 128), lambda i: (i, 0)),
        ],
        out_shape=[jax.ShapeDtypeStruct((N, 64), jnp.float32)] * 4 + [
            jax.ShapeDtypeStruct((N, 128), jnp.float32),
        ],
    )(x, w_aug)


def _ae_body(eat_ref, w1_ref, w2_ref, a1_ref, a2_ref, *, be):
    i = pl.program_id(0)
    blk = eat_ref[...]
    a1_ref[pl.ds(i * be, be)] = jnp.sum(blk * w1_ref[...], axis=0)
    a2_ref[pl.ds(i * be, be)] = jnp.sum(blk * w2_ref[...], axis=0)


def _ae(eat8, we1, we2):
    be = 2048
    grid = (E_PAD // be,)
    return pl.pallas_call(
        functools.partial(_ae_body, be=be),
        grid=grid,
        in_specs=[
            pl.BlockSpec((8, be), lambda i: (0, i)),
            pl.BlockSpec((8, 1), lambda i: (0, 0)),
            pl.BlockSpec((8, 1), lambda i: (0, 0)),
        ],
        out_specs=[
            pl.BlockSpec((E_PAD,), lambda i: (0,)),
            pl.BlockSpec((E_PAD,), lambda i: (0,)),
        ],
        out_shape=[
            jax.ShapeDtypeStruct((E_PAD,), jnp.float32),
            jax.ShapeDtypeStruct((E_PAD,), jnp.float32),
        ],
    )(eat8, we1, we2)


def _kb_body(src_hbm, dst_hbm, ae_hbm, asrc_hbm, adst_hbm,
             ex_hbm, den_hbm,
             src_v, dst_v, ae_v, asrc_v, adst_v, ex_v, den_v):
    wid = lax.axis_index("s") * NC + lax.axis_index("c")
    base = wid * CHUNK_B
    pltpu.sync_copy(src_hbm.at[pl.ds(base, CHUNK_B)], src_v)
    pltpu.sync_copy(dst_hbm.at[pl.ds(base, CHUNK_B)], dst_v)
    pltpu.sync_copy(ae_hbm.at[pl.ds(base, CHUNK_B)], ae_v)
    pltpu.sync_copy(asrc_hbm, asrc_v.at[pl.ds(0, N)])
    pltpu.sync_copy(adst_hbm, adst_v.at[pl.ds(0, N)])

    zf = jnp.zeros((L,), jnp.float32)

    def zero_body(i, _):
        den_v[pl.ds(i * L, L)] = zf
        return _
    lax.fori_loop(0, N_PAD // L, zero_body, None)

    lanes = lax.iota(jnp.int32, L)

    def body(i, _):
        sl = pl.ds(i * L, L)
        sv = src_v[sl]
        dv = dst_v[sl]
        a = plsc.load_gather(asrc_v, [sv]) + plsc.load_gather(adst_v, [dv]) \
            + ae_v[sl]
        a = jnp.maximum(a, 0.2 * a)
        ex = jnp.exp(a)
        gidx = base + i * L + lanes
        ex = jnp.where(gidx < E, ex, 0.0)
        ex_v[sl] = ex
        plsc.addupdate_scatter(den_v, [dv], ex)
        return _
    lax.fori_loop(0, NBLK_B, body, None)

    pltpu.sync_copy(ex_v, ex_hbm.at[pl.ds(base, CHUNK_B)])
    pltpu.sync_copy(den_v, den_hbm.at[wid])


_kb = pl.kernel(
    _kb_body,
    out_type=[
        jax.ShapeDtypeStruct((E_PAD,), jnp.float32),
        jax.ShapeDtypeStruct((NC * NS, N_PAD), jnp.float32),
    ],
    mesh=_mesh,
    compiler_params=pltpu.CompilerParams(needs_layout_passes=False),
    scratch_types=[
        pltpu.VMEM((CHUNK_B,), jnp.int32),
        pltpu.VMEM((CHUNK_B,), jnp.int32),
        pltpu.VMEM((CHUNK_B,), jnp.float32),
        pltpu.VMEM((N_PAD,), jnp.float32),
        pltpu.VMEM((N_PAD,), jnp.float32),
        pltpu.VMEM((CHUNK_B,), jnp.float32),
        pltpu.VMEM((N_PAD,), jnp.float32),
    ],
)


CH_BLKS = 16
N_CHUNK = NBLK_C // CH_BLKS


def _kc_body(hcat_hbm, src_hbm, dst_hbm, ex_hbm, den_hbm, b_hbm,
             out_hbm,
             src_st, dst_st, ex_st, coef_st, den_v, dtmp_v, dsum_v, bias_v,
             rows_v, den_sh, acc_sh):
    c = lax.axis_index("c")
    s = lax.axis_index("s")

    pltpu.sync_copy(b_hbm.at[c], bias_v)

    zf = jnp.zeros((L,), jnp.float32)
    zi = jnp.zeros((L,), jnp.int32)

    nslc = N_PAD // NS
    dslice = pl.ds(s * nslc, nslc)

    def dz_body(i, _):
        dsum_v[pl.ds(i * L, L)] = zf
        return _
    lax.fori_loop(0, nslc // L, dz_body, None)
    for p in range(NC * NS):
        pltpu.sync_copy(den_hbm.at[p, dslice], dtmp_v)

        def dacc_body(i, _):
            sl = pl.ds(i * L, L)
            dsum_v[sl] = dsum_v[sl] + dtmp_v[sl]
            return _
        lax.fori_loop(0, nslc // L, dacc_body, None)
    pltpu.sync_copy(dsum_v, den_sh.at[dslice])
    plsc.subcore_barrier()
    pltpu.sync_copy(den_sh, den_v)

    for q in range(2):
        poff = (2 * c + q) * N

        def rz_body(i, _):
            for g in range(4):
                rows_v[i, pl.ds(g * L, L)] = zf
            return _
        lax.fori_loop(0, ROW_CH, rz_body, None)
        for t in range(ROWS_T // ROW_CH):
            pltpu.sync_copy(
                rows_v, acc_sh.at[pl.ds(s * ROWS_T + t * ROW_CH, ROW_CH)])
        plsc.subcore_barrier()

        def chunk_body(ch, _):
            csl = pl.ds(ch * CH_BLKS, CH_BLKS)
            pltpu.sync_copy(src_hbm.at[s, csl], src_st)
            pltpu.sync_copy(dst_hbm.at[s, csl], dst_st)
            pltpu.sync_copy(ex_hbm.at[s, csl], ex_st)

            def prep_body(i, _):
                for g in range(8):
                    sl = pl.ds(g * L, L)
                    src_st[i, sl] = src_st[i, sl] + poff
                    dn = plsc.load_gather(den_v, [dst_st[i, sl]])
                    coef_st[i, sl] = ex_st[i, sl] / (dn + 1e-16)
                return _
            lax.fori_loop(0, CH_BLKS, prep_body, None)

            def blk_body(j, _):
                pltpu.sync_copy(hcat_hbm.at[src_st.at[j]], rows_v)

                def scale_body(k, _):
                    cv = plsc.load_gather(coef_st.at[j], [zi + k])
                    for g in range(4):
                        sl = pl.ds(g * L, L)
                        rows_v[k, sl] = rows_v[k, sl] * cv
                    return _
                lax.fori_loop(0, 128, scale_body, None)

                pltpu.sync_copy(rows_v, acc_sh.at[dst_st.at[j]], add=True)
                return _
            lax.fori_loop(0, CH_BLKS, blk_body, None)
            return _
        lax.fori_loop(0, N_CHUNK, chunk_body, None)
        plsc.subcore_barrier()

        for t in range(ROWS_T // ROW_CH):
            r0 = s * ROWS_T + t * ROW_CH
            pltpu.sync_copy(acc_sh.at[pl.ds(r0, ROW_CH)], rows_v)

            def bias_body(k, _):
                for g in range(4):
                    rows_v[k, pl.ds(g * L, L)] = (
                        rows_v[k, pl.ds(g * L, L)]
                        + bias_v[pl.ds(q * 64 + g * L, L)])
                return _
            lax.fori_loop(0, ROW_CH, bias_body, None)
            pltpu.sync_copy(rows_v, out_hbm.at[2 * c + q, pl.ds(r0, ROW_CH)])


_kc = pl.kernel(
    _kc_body,
    out_type=jax.ShapeDtypeStruct((2 * NC, N_PAD, 64), jnp.float32),
    mesh=_mesh,
    compiler_params=pltpu.CompilerParams(needs_layout_passes=False,
                                         use_tc_tiling_on_sc=False),
    scratch_types=[
        pltpu.VMEM((CH_BLKS, 128), jnp.int32),
        pltpu.VMEM((CH_BLKS, 128), jnp.int32),
        pltpu.VMEM((CH_BLKS, 128), jnp.float32),
        pltpu.VMEM((CH_BLKS, 128), jnp.float32),
        pltpu.VMEM((N_PAD,), jnp.float32),
        pltpu.VMEM((N_PAD // NS,), jnp.float32),
        pltpu.VMEM((N_PAD // NS,), jnp.float32),
        pltpu.VMEM((HALF,), jnp.float32),
        pltpu.VMEM((ROW_CH, 64), jnp.float32),
        pltpu.VMEM_SHARED((N_PAD,), jnp.float32),
        pltpu.VMEM_SHARED((N_PAD, 64), jnp.float32),
    ],
)


def kernel(x, edge_index, edge_attr,
           W1, att_src1, att_dst1, We1, att_e1, b1,
           W2, att_src2, att_dst2, We2, att_e2, b2):
    src = edge_index[0].astype(jnp.int32)
    dst = edge_index[1].astype(jnp.int32)
    pad = E_PAD - E
    zi = jnp.zeros((pad,), jnp.int32)
    src_p = jnp.concatenate([src, zi])
    dst_p = jnp.concatenate([dst, zi])
    src2d = src_p.reshape(NS, NBLK_C, 128)
    dst2d = dst_p.reshape(NS, NBLK_C, 128)

    eat8 = jnp.zeros((8, E_PAD), jnp.float32).at[:6, :E].set(edge_attr.T)
    we1 = jnp.zeros((8, 1), jnp.float32).at[:6, 0].set(We1 @ att_e1)
    we2 = jnp.zeros((8, 1), jnp.float32).at[:6, 0].set(We2 @ att_e2)
    ae1, ae2 = _ae(eat8, we1, we2)

    def layer(h_in, W, att_s, att_d, ae, bias, relu):
        w_aug = jnp.concatenate(
            [W, (W @ att_s)[:, None], (W @ att_d)[:, None],
             jnp.zeros((W.shape[0], 126), jnp.float32)], axis=1)
        h0, h1, h2, h3, aug = _dense(h_in, w_aug, relu)
        hcat = jnp.concatenate([h0, h1, h2, h3], axis=0)
        ex, den = _kb(src_p, dst_p, ae, aug[:, 0], aug[:, 1])
        ex2d = ex.reshape(NS, NBLK_C, 128)
        o = _kc(hcat, src2d, dst2d, ex2d, den, bias.reshape(NC, HALF))
        return jnp.concatenate([o[0], o[1], o[2], o[3]], axis=1)[:N]

    o1 = layer(x, W1, att_src1, att_dst1, ae1, b1, relu=False)
    o2 = layer(o1, W2, att_src2, att_dst2, ae2, b2, relu=True)
    return o2


# speedup vs baseline: 12.4173x; 1.5684x over previous
import functools

import jax
import jax.numpy as jnp
from jax import lax
from jax.experimental import pallas as pl
from jax.experimental.pallas import tpu as pltpu
from jax.experimental.pallas import tpu_sc as plsc

N = 10000
E = 320000
IN_DIM = 128
HID = 256
HALF = 128
NC = 2
NS = 16
L = 16

E_PAD = 327680
CHUNK_B = E_PAD // (NC * NS)
NBLK_B = CHUNK_B // L
CHUNK_C = E_PAD // NS
NBLK_C = CHUNK_C // 128
N_PAD = 10240
ROWS_T = N_PAD // NS
ROW_CH = 128

_mesh = plsc.VectorSubcoreMesh(core_axis_name="c", subcore_axis_name="s")


def _dense_body(x_ref, w_ref, ha0, ha1, ha2, ha3, aug_ref, *, relu):
    ha_ref = (ha0, ha1, ha2, ha3)
    xb = x_ref[...]
    if relu:
        xb = jnp.maximum(xb, 0.0)
    h = jnp.dot(xb, w_ref[...], preferred_element_type=jnp.float32)
    for q in range(4):
        ha_ref[q][...] = h[:, q * 64:(q + 1) * 64]
    aug_ref[...] = h[:, HID:]


def _dense(x, w_aug, relu):
    rows = 1000
    grid = (N // rows,)
    in_dim = x.shape[1]
    return pl.pallas_call(
        functools.partial(_dense_body, relu=relu),
        grid=grid,
        in_specs=[
            pl.BlockSpec((rows, in_dim), lambda i: (i, 0)),
            pl.BlockSpec((in_dim, HID + 128), lambda i: (0, 0)),
        ],
        out_specs=[pl.BlockSpec((rows, 64), lambda i: (i, 0))] * 4 + [
            pl.BlockSpec((rows, 128), lambda i: (i, 0)),
        ],
        out_shape=[jax.ShapeDtypeStruct((N, 64), jnp.float32)] * 4 + [
            jax.ShapeDtypeStruct((N, 128), jnp.float32),
        ],
    )(x, w_aug)


def _ae_body(eat_ref, w1_ref, w2_ref, a1_ref, a2_ref, *, be):
    i = pl.program_id(0)
    blk = eat_ref[...]
    a1_ref[pl.ds(i * be, be)] = jnp.sum(blk * w1_ref[...], axis=0)
    a2_ref[pl.ds(i * be, be)] = jnp.sum(blk * w2_ref[...], axis=0)


def _ae(eat8, we1, we2):
    be = 2048
    grid = (E_PAD // be,)
    return pl.pallas_call(
        functools.partial(_ae_body, be=be),
        grid=grid,
        in_specs=[
            pl.BlockSpec((8, be), lambda i: (0, i)),
            pl.BlockSpec((8, 1), lambda i: (0, 0)),
            pl.BlockSpec((8, 1), lambda i: (0, 0)),
        ],
        out_specs=[
            pl.BlockSpec((E_PAD,), lambda i: (0,)),
            pl.BlockSpec((E_PAD,), lambda i: (0,)),
        ],
        out_shape=[
            jax.ShapeDtypeStruct((E_PAD,), jnp.float32),
            jax.ShapeDtypeStruct((E_PAD,), jnp.float32),
        ],
    )(eat8, we1, we2)


def _kb_body(src_hbm, dst_hbm, ae_hbm, asrc_hbm, adst_hbm,
             ex_hbm, den_hbm,
             src_v, dst_v, ae_v, asrc_v, adst_v, ex_v, den_v):
    wid = lax.axis_index("s") * NC + lax.axis_index("c")
    base = wid * CHUNK_B
    pltpu.sync_copy(src_hbm.at[pl.ds(base, CHUNK_B)], src_v)
    pltpu.sync_copy(dst_hbm.at[pl.ds(base, CHUNK_B)], dst_v)
    pltpu.sync_copy(ae_hbm.at[pl.ds(base, CHUNK_B)], ae_v)
    pltpu.sync_copy(asrc_hbm, asrc_v.at[pl.ds(0, N)])
    pltpu.sync_copy(adst_hbm, adst_v.at[pl.ds(0, N)])

    zf = jnp.zeros((L,), jnp.float32)

    def zero_body(i, _):
        den_v[pl.ds(i * L, L)] = zf
        return _
    lax.fori_loop(0, N_PAD // L, zero_body, None)

    lanes = lax.iota(jnp.int32, L)

    def body(i, _):
        sl = pl.ds(i * L, L)
        sv = src_v[sl]
        dv = dst_v[sl]
        a = plsc.load_gather(asrc_v, [sv]) + plsc.load_gather(adst_v, [dv]) \
            + ae_v[sl]
        a = jnp.maximum(a, 0.2 * a)
        ex = jnp.exp(a)
        gidx = base + i * L + lanes
        ex = jnp.where(gidx < E, ex, 0.0)
        ex_v[sl] = ex
        plsc.addupdate_scatter(den_v, [dv], ex)
        return _
    lax.fori_loop(0, NBLK_B, body, None)

    pltpu.sync_copy(ex_v, ex_hbm.at[pl.ds(base, CHUNK_B)])
    pltpu.sync_copy(den_v, den_hbm.at[wid])


_kb = pl.kernel(
    _kb_body,
    out_type=[
        jax.ShapeDtypeStruct((E_PAD,), jnp.float32),
        jax.ShapeDtypeStruct((NC * NS, N_PAD), jnp.float32),
    ],
    mesh=_mesh,
    compiler_params=pltpu.CompilerParams(needs_layout_passes=False),
    scratch_types=[
        pltpu.VMEM((CHUNK_B,), jnp.int32),
        pltpu.VMEM((CHUNK_B,), jnp.int32),
        pltpu.VMEM((CHUNK_B,), jnp.float32),
        pltpu.VMEM((N_PAD,), jnp.float32),
        pltpu.VMEM((N_PAD,), jnp.float32),
        pltpu.VMEM((CHUNK_B,), jnp.float32),
        pltpu.VMEM((N_PAD,), jnp.float32),
    ],
)


def _kd_body(dst_hbm, ex_hbm, den_hbm, coef_hbm,
             dst_v, ex_v, den_v, dtmp_v, dsum_v, den_sh):
    c = lax.axis_index("c")
    s = lax.axis_index("s")
    wid = s * NC + c
    base = wid * CHUNK_B

    pltpu.sync_copy(dst_hbm.at[pl.ds(base, CHUNK_B)], dst_v)
    pltpu.sync_copy(ex_hbm.at[pl.ds(base, CHUNK_B)], ex_v)

    nslc = N_PAD // NS
    dslice = pl.ds(s * nslc, nslc)
    zf = jnp.zeros((L,), jnp.float32)

    def dz_body(i, _):
        dsum_v[pl.ds(i * L, L)] = zf
        return _
    lax.fori_loop(0, nslc // L, dz_body, None)
    for p in range(NC * NS):
        pltpu.sync_copy(den_hbm.at[p, dslice], dtmp_v)

        def dacc_body(i, _):
            sl = pl.ds(i * L, L)
            dsum_v[sl] = dsum_v[sl] + dtmp_v[sl]
            return _
        lax.fori_loop(0, nslc // L, dacc_body, None)
    pltpu.sync_copy(dsum_v, den_sh.at[dslice])
    plsc.subcore_barrier()
    pltpu.sync_copy(den_sh, den_v)

    def body(i, _):
        sl = pl.ds(i * L, L)
        dn = plsc.load_gather(den_v, [dst_v[sl]])
        ex_v[sl] = ex_v[sl] / (dn + 1e-16)
        return _
    lax.fori_loop(0, NBLK_B, body, None)

    pltpu.sync_copy(ex_v, coef_hbm.at[pl.ds(base, CHUNK_B)])


_kd = pl.kernel(
    _kd_body,
    out_type=jax.ShapeDtypeStruct((E_PAD,), jnp.float32),
    mesh=_mesh,
    compiler_params=pltpu.CompilerParams(needs_layout_passes=False),
    scratch_types=[
        pltpu.VMEM((CHUNK_B,), jnp.int32),
        pltpu.VMEM((CHUNK_B,), jnp.float32),
        pltpu.VMEM((N_PAD,), jnp.float32),
        pltpu.VMEM((N_PAD // NS,), jnp.float32),
        pltpu.VMEM((N_PAD // NS,), jnp.float32),
        pltpu.VMEM_SHARED((N_PAD,), jnp.float32),
    ],
)


CB = 32
N_CHUNK = NBLK_C // CB
NBUF = 3


def _kc_body(hcat_hbm, epk_hbm, b_hbm,
             out_hbm,
             st_v, bias_v, r0_v, r1_v, r2_v, gsem, ssem, acc_sh):
    c = lax.axis_index("c")
    s = lax.axis_index("s")
    rows = (r0_v, r1_v, r2_v)

    pltpu.sync_copy(b_hbm.at[c], bias_v)

    zf = jnp.zeros((L,), jnp.float32)
    zi = jnp.zeros((L,), jnp.int32)

    for q in range(2):
        poff = (2 * c + q) * N

        def rz_body(i, _):
            for g in range(4):
                r0_v[i, pl.ds(g * L, L)] = zf
            return _
        lax.fori_loop(0, ROW_CH, rz_body, None)
        for t in range(ROWS_T // ROW_CH):
            pltpu.sync_copy(
                r0_v, acc_sh.at[pl.ds(s * ROWS_T + t * ROW_CH, ROW_CH)])
        plsc.subcore_barrier()

        def chunk_body(ch, _):
            pltpu.sync_copy(epk_hbm.at[s, ch], st_v)

            def prep_body(i, _):
                for g in range(8):
                    sl = pl.ds(g * L, L)
                    st_v[0, i, sl] = st_v[0, i, sl] + poff
                return _
            lax.fori_loop(0, CB, prep_body, None)

            def scale(b, j):
                cf_row = st_v.at[2, j]

                def scale_body(k, _):
                    cv = plsc.bitcast(
                        plsc.load_gather(cf_row, [zi + 2 * k]), jnp.float32)
                    cw = plsc.bitcast(
                        plsc.load_gather(cf_row, [zi + 2 * k + 1]),
                        jnp.float32)
                    for g in range(4):
                        sl = pl.ds(g * L, L)
                        rows[b][2 * k, sl] = rows[b][2 * k, sl] * cv
                        rows[b][2 * k + 1, sl] = rows[b][2 * k + 1, sl] * cw
                    return _
                lax.fori_loop(0, 64, scale_body, None)

            def issue_gather(j):
                return pltpu.async_copy(
                    hcat_hbm.at[st_v.at[0, j]], rows[j % NBUF],
                    gsem.at[j % NBUF])

            gd = {}
            sd = {}
            for j in range(min(2, CB)):
                gd[j] = issue_gather(j)
            for j in range(CB):
                b = j % NBUF
                gd[j].wait()
                scale(b, j)
                sd[j] = pltpu.async_copy(
                    rows[b], acc_sh.at[st_v.at[1, j]], ssem.at[b], add=True)
                nj = j + 2
                if nj < CB:
                    if nj >= NBUF:
                        sd[nj - NBUF].wait()
                    gd[nj] = issue_gather(nj)
            for j in range(CB - NBUF, CB):
                sd[j].wait()
            return _
        lax.fori_loop(0, N_CHUNK, chunk_body, None)
        plsc.subcore_barrier()

        for t in range(ROWS_T // ROW_CH):
            r0 = s * ROWS_T + t * ROW_CH
            pltpu.sync_copy(acc_sh.at[pl.ds(r0, ROW_CH)], r0_v)

            def bias_body(k, _):
                for g in range(4):
                    r0_v[k, pl.ds(g * L, L)] = (
                        r0_v[k, pl.ds(g * L, L)]
                        + bias_v[pl.ds(q * 64 + g * L, L)])
                return _
            lax.fori_loop(0, ROW_CH, bias_body, None)
            pltpu.sync_copy(r0_v, out_hbm.at[2 * c + q, pl.ds(r0, ROW_CH)])


_kc = pl.kernel(
    _kc_body,
    out_type=jax.ShapeDtypeStruct((2 * NC, N_PAD, 64), jnp.float32),
    mesh=_mesh,
    compiler_params=pltpu.CompilerParams(needs_layout_passes=False,
                                         use_tc_tiling_on_sc=False),
    scratch_types=[
        pltpu.VMEM((3, CB, 128), jnp.int32),
        pltpu.VMEM((HALF,), jnp.float32),
        pltpu.VMEM((ROW_CH, 64), jnp.float32),
        pltpu.VMEM((ROW_CH, 64), jnp.float32),
        pltpu.VMEM((ROW_CH, 64), jnp.float32),
        pltpu.SemaphoreType.DMA((NBUF,)),
        pltpu.SemaphoreType.DMA((NBUF,)),
        pltpu.VMEM_SHARED((N_PAD, 64), jnp.float32),
    ],
)


def kernel(x, edge_index, edge_attr,
           W1, att_src1, att_dst1, We1, att_e1, b1,
           W2, att_src2, att_dst2, We2, att_e2, b2):
    src = edge_index[0].astype(jnp.int32)
    dst = edge_index[1].astype(jnp.int32)
    pad = E_PAD - E
    zi = jnp.zeros((pad,), jnp.int32)
    src_p = jnp.concatenate([src, zi])
    dst_p = jnp.concatenate([dst, zi])
    src5 = src_p.reshape(NS, N_CHUNK, CB, 128)
    dst5 = dst_p.reshape(NS, N_CHUNK, CB, 128)

    eat8 = jnp.zeros((8, E_PAD), jnp.float32).at[:6, :E].set(edge_attr.T)
    we1 = jnp.zeros((8, 1), jnp.float32).at[:6, 0].set(We1 @ att_e1)
    we2 = jnp.zeros((8, 1), jnp.float32).at[:6, 0].set(We2 @ att_e2)
    ae1, ae2 = _ae(eat8, we1, we2)

    def layer(h_in, W, att_s, att_d, ae, bias, relu):
        w_aug = jnp.concatenate(
            [W, (W @ att_s)[:, None], (W @ att_d)[:, None],
             jnp.zeros((W.shape[0], 126), jnp.float32)], axis=1)
        h0, h1, h2, h3, aug = _dense(h_in, w_aug, relu)
        hcat = jnp.concatenate([h0, h1, h2, h3], axis=0)
        ex, den = _kb(src_p, dst_p, ae, aug[:, 0], aug[:, 1])
        coef = _kd(dst_p, ex, den)
        cf5 = lax.bitcast_convert_type(
            coef.reshape(NS, N_CHUNK, CB, 128), jnp.int32)
        epk = jnp.stack([src5, dst5, cf5], axis=2)
        o = _kc(hcat, epk, bias.reshape(NC, HALF))
        return jnp.concatenate([o[0], o[1], o[2], o[3]], axis=1)[:N]

    o1 = layer(x, W1, att_src1, att_dst1, ae1, b1, relu=False)
    o2 = layer(o1, W2, att_src2, att_dst2, ae2, b2, relu=True)
    return o2


# speedup vs baseline: 12.7352x; 1.0256x over previous
import functools

import jax
import jax.numpy as jnp
from jax import lax
from jax.experimental import pallas as pl
from jax.experimental.pallas import tpu as pltpu
from jax.experimental.pallas import tpu_sc as plsc

N = 10000
E = 320000
IN_DIM = 128
HID = 256
HALF = 128
NC = 2
NS = 16
L = 16

E_PAD = 327680
CHUNK_B = E_PAD // (NC * NS)
NBLK_B = CHUNK_B // L
CHUNK_C = E_PAD // NS
NBLK_C = CHUNK_C // 128
N_PAD = 10240
ROWS_T = N_PAD // NS
ROW_CH = 128

_mesh = plsc.VectorSubcoreMesh(core_axis_name="c", subcore_axis_name="s")


def _dense_body(x_ref, w_ref, ha0, ha1, ha2, ha3, aug_ref, *, relu):
    ha_ref = (ha0, ha1, ha2, ha3)
    xb = x_ref[...]
    if xb.ndim == 3:
        xb = jnp.concatenate([xb[0], xb[1], xb[2], xb[3]], axis=1)
    if relu:
        xb = jnp.maximum(xb, 0.0)
    h = jnp.dot(xb, w_ref[...], preferred_element_type=jnp.float32)
    for q in range(4):
        ha_ref[q][...] = h[:, q * 64:(q + 1) * 64]
    aug_ref[...] = h[:, HID:]


def _dense(x, w_aug, relu):
    rows = 1280
    grid = (N_PAD // rows,)
    in_dim = w_aug.shape[0]
    if x.ndim == 3:
        in_spec = pl.BlockSpec((4, rows, 64), lambda i: (0, i, 0))
    else:
        in_spec = pl.BlockSpec((rows, in_dim), lambda i: (i, 0))
    return pl.pallas_call(
        functools.partial(_dense_body, relu=relu),
        grid=grid,
        in_specs=[
            in_spec,
            pl.BlockSpec((in_dim, HID + 128), lambda i: (0, 0)),
        ],
        out_specs=[pl.BlockSpec((rows, 64), lambda i: (i, 0))] * 4 + [
            pl.BlockSpec((rows, 128), lambda i: (i, 0)),
        ],
        out_shape=[jax.ShapeDtypeStruct((N_PAD, 64), jnp.float32)] * 4 + [
            jax.ShapeDtypeStruct((N_PAD, 128), jnp.float32),
        ],
    )(x, w_aug)


def _ae_body(eat_ref, w1_ref, w2_ref, a1_ref, a2_ref, *, be):
    i = pl.program_id(0)
    blk = eat_ref[...]
    a1_ref[pl.ds(i * be, be)] = jnp.sum(blk * w1_ref[...], axis=0)
    a2_ref[pl.ds(i * be, be)] = jnp.sum(blk * w2_ref[...], axis=0)


def _ae(eat8, we1, we2):
    be = 2048
    grid = (E_PAD // be,)
    return pl.pallas_call(
        functools.partial(_ae_body, be=be),
        grid=grid,
        in_specs=[
            pl.BlockSpec((8, be), lambda i: (0, i)),
            pl.BlockSpec((8, 1), lambda i: (0, 0)),
            pl.BlockSpec((8, 1), lambda i: (0, 0)),
        ],
        out_specs=[
            pl.BlockSpec((E_PAD,), lambda i: (0,)),
            pl.BlockSpec((E_PAD,), lambda i: (0,)),
        ],
        out_shape=[
            jax.ShapeDtypeStruct((E_PAD,), jnp.float32),
            jax.ShapeDtypeStruct((E_PAD,), jnp.float32),
        ],
    )(eat8, we1, we2)


def _kb_body(src_hbm, dst_hbm, ae_hbm, asrc_hbm, adst_hbm,
             ex_hbm, den_hbm,
             src_v, dst_v, ae_v, asrc_v, adst_v, ex_v, den_v):
    wid = lax.axis_index("s") * NC + lax.axis_index("c")
    base = wid * CHUNK_B
    pltpu.sync_copy(src_hbm.at[pl.ds(base, CHUNK_B)], src_v)
    pltpu.sync_copy(dst_hbm.at[pl.ds(base, CHUNK_B)], dst_v)
    pltpu.sync_copy(ae_hbm.at[pl.ds(base, CHUNK_B)], ae_v)
    pltpu.sync_copy(asrc_hbm, asrc_v)
    pltpu.sync_copy(adst_hbm, adst_v)

    zf = jnp.zeros((L,), jnp.float32)

    def zero_body(i, _):
        den_v[pl.ds(i * L, L)] = zf
        return _
    lax.fori_loop(0, N_PAD // L, zero_body, None)

    lanes = lax.iota(jnp.int32, L)

    def body(i, _):
        sl = pl.ds(i * L, L)
        sv = src_v[sl]
        dv = dst_v[sl]
        a = plsc.load_gather(asrc_v, [sv]) + plsc.load_gather(adst_v, [dv]) \
            + ae_v[sl]
        a = jnp.maximum(a, 0.2 * a)
        ex = jnp.exp(a)
        gidx = base + i * L + lanes
        ex = jnp.where(gidx < E, ex, 0.0)
        ex_v[sl] = ex
        plsc.addupdate_scatter(den_v, [dv], ex)
        return _
    lax.fori_loop(0, NBLK_B, body, None)

    pltpu.sync_copy(ex_v, ex_hbm.at[pl.ds(base, CHUNK_B)])
    pltpu.sync_copy(den_v, den_hbm.at[wid])


_kb = pl.kernel(
    _kb_body,
    out_type=[
        jax.ShapeDtypeStruct((E_PAD,), jnp.float32),
        jax.ShapeDtypeStruct((NC * NS, N_PAD), jnp.float32),
    ],
    mesh=_mesh,
    compiler_params=pltpu.CompilerParams(needs_layout_passes=False),
    scratch_types=[
        pltpu.VMEM((CHUNK_B,), jnp.int32),
        pltpu.VMEM((CHUNK_B,), jnp.int32),
        pltpu.VMEM((CHUNK_B,), jnp.float32),
        pltpu.VMEM((N_PAD,), jnp.float32),
        pltpu.VMEM((N_PAD,), jnp.float32),
        pltpu.VMEM((CHUNK_B,), jnp.float32),
        pltpu.VMEM((N_PAD,), jnp.float32),
    ],
)


def _kd_body(dst_hbm, ex_hbm, den_hbm, coef_hbm,
             dst_v, ex_v, den_v, dtmp_v, dsum_v, den_sh):
    c = lax.axis_index("c")
    s = lax.axis_index("s")
    wid = s * NC + c
    base = wid * CHUNK_B

    pltpu.sync_copy(dst_hbm.at[pl.ds(base, CHUNK_B)], dst_v)
    pltpu.sync_copy(ex_hbm.at[pl.ds(base, CHUNK_B)], ex_v)

    nslc = N_PAD // NS
    dslice = pl.ds(s * nslc, nslc)
    zf = jnp.zeros((L,), jnp.float32)

    def dz_body(i, _):
        dsum_v[pl.ds(i * L, L)] = zf
        return _
    lax.fori_loop(0, nslc // L, dz_body, None)
    for p in range(NC * NS):
        pltpu.sync_copy(den_hbm.at[p, dslice], dtmp_v)

        def dacc_body(i, _):
            sl = pl.ds(i * L, L)
            dsum_v[sl] = dsum_v[sl] + dtmp_v[sl]
            return _
        lax.fori_loop(0, nslc // L, dacc_body, None)
    pltpu.sync_copy(dsum_v, den_sh.at[dslice])
    plsc.subcore_barrier()
    pltpu.sync_copy(den_sh, den_v)

    def body(i, _):
        sl = pl.ds(i * L, L)
        dn = plsc.load_gather(den_v, [dst_v[sl]])
        ex_v[sl] = ex_v[sl] / (dn + 1e-16)
        return _
    lax.fori_loop(0, NBLK_B, body, None)

    pltpu.sync_copy(ex_v, coef_hbm.at[pl.ds(base, CHUNK_B)])


_kd = pl.kernel(
    _kd_body,
    out_type=jax.ShapeDtypeStruct((E_PAD,), jnp.float32),
    mesh=_mesh,
    compiler_params=pltpu.CompilerParams(needs_layout_passes=False),
    scratch_types=[
        pltpu.VMEM((CHUNK_B,), jnp.int32),
        pltpu.VMEM((CHUNK_B,), jnp.float32),
        pltpu.VMEM((N_PAD,), jnp.float32),
        pltpu.VMEM((N_PAD // NS,), jnp.float32),
        pltpu.VMEM((N_PAD // NS,), jnp.float32),
        pltpu.VMEM_SHARED((N_PAD,), jnp.float32),
    ],
)


CB = 32
N_CHUNK = NBLK_C // CB
NBUF = 3


def _kc_body(h0_hbm, h1_hbm, h2_hbm, h3_hbm, epk_hbm, b_hbm,
             out_hbm,
             st_v, bias_v, r0_v, r1_v, r2_v, gsem, ssem, acc_sh):
    c = lax.axis_index("c")
    s = lax.axis_index("s")
    rows = (r0_v, r1_v, r2_v)
    hp = (h0_hbm, h1_hbm, h2_hbm, h3_hbm)

    pltpu.sync_copy(b_hbm.at[c], bias_v)

    zf = jnp.zeros((L,), jnp.float32)
    zi = jnp.zeros((L,), jnp.int32)

    for q in range(2):

        def rz_body(i, _):
            for g in range(4):
                r0_v[i, pl.ds(g * L, L)] = zf
            return _
        lax.fori_loop(0, ROW_CH, rz_body, None)
        for t in range(ROWS_T // ROW_CH):
            pltpu.sync_copy(
                r0_v, acc_sh.at[pl.ds(s * ROWS_T + t * ROW_CH, ROW_CH)])
        plsc.subcore_barrier()

        def chunk_body(ch, _):
            pltpu.sync_copy(epk_hbm.at[s, ch], st_v)

            def scale(b, j):
                cf_row = st_v.at[2, j]

                def scale_body(k, _):
                    cv = plsc.bitcast(
                        plsc.load_gather(cf_row, [zi + 2 * k]), jnp.float32)
                    cw = plsc.bitcast(
                        plsc.load_gather(cf_row, [zi + 2 * k + 1]),
                        jnp.float32)
                    for g in range(4):
                        sl = pl.ds(g * L, L)
                        rows[b][2 * k, sl] = rows[b][2 * k, sl] * cv
                        rows[b][2 * k + 1, sl] = rows[b][2 * k + 1, sl] * cw
                    return _
                lax.fori_loop(0, 64, scale_body, None)

            def issue_gather(j):
                idx = st_v.at[0, j]
                buf = rows[j % NBUF]
                sem = gsem.at[j % NBUF]
                ds = [pltpu.make_async_copy(hp[2 * cc + q].at[idx], buf, sem)
                      for cc in range(NC)]
                for cc in range(NC):
                    pl.when(c == cc)(ds[cc].start)
                return ds

            def wait_gather(ds):
                ds[0].wait()

            gd = {}
            sd = {}
            for j in range(min(2, CB)):
                gd[j] = issue_gather(j)
            for j in range(CB):
                b = j % NBUF
                wait_gather(gd[j])
                scale(b, j)
                sd[j] = pltpu.async_copy(
                    rows[b], acc_sh.at[st_v.at[1, j]], ssem.at[b], add=True)
                nj = j + 2
                if nj < CB:
                    if nj >= NBUF:
                        sd[nj - NBUF].wait()
                    gd[nj] = issue_gather(nj)
            for j in range(CB - NBUF, CB):
                sd[j].wait()
            return _
        lax.fori_loop(0, N_CHUNK, chunk_body, None)
        plsc.subcore_barrier()

        for t in range(ROWS_T // ROW_CH):
            r0 = s * ROWS_T + t * ROW_CH
            pltpu.sync_copy(acc_sh.at[pl.ds(r0, ROW_CH)], r0_v)

            def bias_body(k, _):
                for g in range(4):
                    r0_v[k, pl.ds(g * L, L)] = (
                        r0_v[k, pl.ds(g * L, L)]
                        + bias_v[pl.ds(q * 64 + g * L, L)])
                return _
            lax.fori_loop(0, ROW_CH, bias_body, None)
            pltpu.sync_copy(r0_v, out_hbm.at[2 * c + q, pl.ds(r0, ROW_CH)])


_kc = pl.kernel(
    _kc_body,
    out_type=jax.ShapeDtypeStruct((2 * NC, N_PAD, 64), jnp.float32),
    mesh=_mesh,
    compiler_params=pltpu.CompilerParams(needs_layout_passes=False,
                                         use_tc_tiling_on_sc=False),
    scratch_types=[
        pltpu.VMEM((3, CB, 128), jnp.int32),
        pltpu.VMEM((HALF,), jnp.float32),
        pltpu.VMEM((ROW_CH, 64), jnp.float32),
        pltpu.VMEM((ROW_CH, 64), jnp.float32),
        pltpu.VMEM((ROW_CH, 64), jnp.float32),
        pltpu.SemaphoreType.DMA((NBUF,)),
        pltpu.SemaphoreType.DMA((NBUF,)),
        pltpu.VMEM_SHARED((N_PAD, 64), jnp.float32),
    ],
)


def kernel(x, edge_index, edge_attr,
           W1, att_src1, att_dst1, We1, att_e1, b1,
           W2, att_src2, att_dst2, We2, att_e2, b2):
    src = edge_index[0].astype(jnp.int32)
    dst = edge_index[1].astype(jnp.int32)
    pad = E_PAD - E
    zi = jnp.zeros((pad,), jnp.int32)
    src_p = jnp.concatenate([src, zi])
    dst_p = jnp.concatenate([dst, zi])
    src5 = src_p.reshape(NS, N_CHUNK, CB, 128)
    dst5 = dst_p.reshape(NS, N_CHUNK, CB, 128)

    eat8 = jnp.zeros((8, E_PAD), jnp.float32).at[:6, :E].set(edge_attr.T)
    we1 = jnp.zeros((8, 1), jnp.float32).at[:6, 0].set(We1 @ att_e1)
    we2 = jnp.zeros((8, 1), jnp.float32).at[:6, 0].set(We2 @ att_e2)
    ae1, ae2 = _ae(eat8, we1, we2)

    def layer(h_in, W, att_s, att_d, ae, bias, relu):
        w_aug = jnp.concatenate(
            [W, (W @ att_s)[:, None], (W @ att_d)[:, None],
             jnp.zeros((W.shape[0], 126), jnp.float32)], axis=1)
        h0, h1, h2, h3, aug = _dense(h_in, w_aug, relu)
        ex, den = _kb(src_p, dst_p, ae, aug[:, 0], aug[:, 1])
        coef = _kd(dst_p, ex, den)
        cf5 = lax.bitcast_convert_type(
            coef.reshape(NS, N_CHUNK, CB, 128), jnp.int32)
        epk = jnp.stack([src5, dst5, cf5], axis=2)
        return _kc(h0, h1, h2, h3, epk, bias.reshape(NC, HALF))

    x_pad = jnp.zeros((N_PAD, IN_DIM), jnp.float32).at[:N].set(x)
    o1 = layer(x_pad, W1, att_src1, att_dst1, ae1, b1, relu=False)
    o2 = layer(o1, W2, att_src2, att_dst2, ae2, b2, relu=True)
    return jnp.concatenate([o2[0], o2[1], o2[2], o2[3]], axis=1)[:N]


# speedup vs baseline: 13.0552x; 1.0251x over previous
import functools

import jax
import jax.numpy as jnp
from jax import lax
from jax.experimental import pallas as pl
from jax.experimental.pallas import tpu as pltpu
from jax.experimental.pallas import tpu_sc as plsc

N = 10000
E = 320000
IN_DIM = 128
HID = 256
HALF = 128
NC = 2
NS = 16
L = 16

E_PAD = 327680
CHUNK_B = E_PAD // (NC * NS)
NBLK_B = CHUNK_B // L
CHUNK_C = E_PAD // NS
NBLK_C = CHUNK_C // 128
N_PAD = 10240
ROWS_T = N_PAD // NS
ROW_CH = 128

_mesh = plsc.VectorSubcoreMesh(core_axis_name="c", subcore_axis_name="s")


def _dense_body(x_ref, w_ref, ha0, ha1, ha2, ha3, aug_ref, *, relu):
    ha_ref = (ha0, ha1, ha2, ha3)
    xb = x_ref[...]
    if xb.ndim == 3:
        xb = jnp.concatenate([xb[0], xb[1], xb[2], xb[3]], axis=1)
    if relu:
        xb = jnp.maximum(xb, 0.0)
    h = jnp.dot(xb, w_ref[...], preferred_element_type=jnp.float32)
    for q in range(4):
        ha_ref[q][...] = h[:, q * 64:(q + 1) * 64]
    aug_ref[...] = h[:, HID:]


def _dense(x, w_aug, relu):
    rows = 1280
    grid = (N_PAD // rows,)
    in_dim = w_aug.shape[0]
    if x.ndim == 3:
        in_spec = pl.BlockSpec((4, rows, 64), lambda i: (0, i, 0))
    else:
        in_spec = pl.BlockSpec((rows, in_dim), lambda i: (i, 0))
    return pl.pallas_call(
        functools.partial(_dense_body, relu=relu),
        grid=grid,
        in_specs=[
            in_spec,
            pl.BlockSpec((in_dim, HID + 128), lambda i: (0, 0)),
        ],
        out_specs=[pl.BlockSpec((rows, 64), lambda i: (i, 0))] * 4 + [
            pl.BlockSpec((rows, 128), lambda i: (i, 0)),
        ],
        out_shape=[jax.ShapeDtypeStruct((N_PAD, 64), jnp.float32)] * 4 + [
            jax.ShapeDtypeStruct((N_PAD, 128), jnp.float32),
        ],
    )(x, w_aug)


def _ae_body(eat_ref, w1_ref, w2_ref, a1_ref, a2_ref, *, be):
    i = pl.program_id(0)
    blk = eat_ref[...]
    a1_ref[pl.ds(i * be, be)] = jnp.sum(blk * w1_ref[...], axis=0)
    a2_ref[pl.ds(i * be, be)] = jnp.sum(blk * w2_ref[...], axis=0)


def _ae(eat8, we1, we2):
    be = 2048
    grid = (E_PAD // be,)
    return pl.pallas_call(
        functools.partial(_ae_body, be=be),
        grid=grid,
        in_specs=[
            pl.BlockSpec((8, be), lambda i: (0, i)),
            pl.BlockSpec((8, 1), lambda i: (0, 0)),
            pl.BlockSpec((8, 1), lambda i: (0, 0)),
        ],
        out_specs=[
            pl.BlockSpec((E_PAD,), lambda i: (0,)),
            pl.BlockSpec((E_PAD,), lambda i: (0,)),
        ],
        out_shape=[
            jax.ShapeDtypeStruct((E_PAD,), jnp.float32),
            jax.ShapeDtypeStruct((E_PAD,), jnp.float32),
        ],
    )(eat8, we1, we2)


def _kb_body(src_hbm, dst_hbm, ae_hbm, asrc_hbm, adst_hbm,
             ex_hbm, den_hbm,
             src_v, dst_v, ae_v, asrc_v, adst_v, ex_v, den_v):
    wid = lax.axis_index("s") * NC + lax.axis_index("c")
    base = wid * CHUNK_B
    pltpu.sync_copy(src_hbm.at[pl.ds(base, CHUNK_B)], src_v)
    pltpu.sync_copy(dst_hbm.at[pl.ds(base, CHUNK_B)], dst_v)
    pltpu.sync_copy(ae_hbm.at[pl.ds(base, CHUNK_B)], ae_v)
    pltpu.sync_copy(asrc_hbm, asrc_v)
    pltpu.sync_copy(adst_hbm, adst_v)

    zf = jnp.zeros((L,), jnp.float32)

    def zero_body(i, _):
        den_v[pl.ds(i * L, L)] = zf
        return _
    lax.fori_loop(0, N_PAD // L, zero_body, None)

    lanes = lax.iota(jnp.int32, L)

    def body(i, _):
        sl = pl.ds(i * L, L)
        sv = src_v[sl]
        dv = dst_v[sl]
        a = plsc.load_gather(asrc_v, [sv]) + plsc.load_gather(adst_v, [dv]) \
            + ae_v[sl]
        a = jnp.maximum(a, 0.2 * a)
        ex = jnp.exp(a)
        gidx = base + i * L + lanes
        ex = jnp.where(gidx < E, ex, 0.0)
        ex_v[sl] = ex
        plsc.addupdate_scatter(den_v, [dv], ex)
        return _
    lax.fori_loop(0, NBLK_B, body, None)

    pltpu.sync_copy(ex_v, ex_hbm.at[pl.ds(base, CHUNK_B)])
    pltpu.sync_copy(den_v, den_hbm.at[wid])


_kb = pl.kernel(
    _kb_body,
    out_type=[
        jax.ShapeDtypeStruct((E_PAD,), jnp.float32),
        jax.ShapeDtypeStruct((NC * NS, N_PAD), jnp.float32),
    ],
    mesh=_mesh,
    compiler_params=pltpu.CompilerParams(needs_layout_passes=False),
    scratch_types=[
        pltpu.VMEM((CHUNK_B,), jnp.int32),
        pltpu.VMEM((CHUNK_B,), jnp.int32),
        pltpu.VMEM((CHUNK_B,), jnp.float32),
        pltpu.VMEM((N_PAD,), jnp.float32),
        pltpu.VMEM((N_PAD,), jnp.float32),
        pltpu.VMEM((CHUNK_B,), jnp.float32),
        pltpu.VMEM((N_PAD,), jnp.float32),
    ],
)


def _kd_body(dst_hbm, ex_hbm, den_hbm, coef_hbm,
             dst_v, ex_v, den_v, dtmp_v, dsum_v, den_sh):
    c = lax.axis_index("c")
    s = lax.axis_index("s")
    wid = s * NC + c
    base = wid * CHUNK_B

    pltpu.sync_copy(dst_hbm.at[pl.ds(base, CHUNK_B)], dst_v)
    pltpu.sync_copy(ex_hbm.at[pl.ds(base, CHUNK_B)], ex_v)

    nslc = N_PAD // NS
    dslice = pl.ds(s * nslc, nslc)
    zf = jnp.zeros((L,), jnp.float32)

    def dz_body(i, _):
        dsum_v[pl.ds(i * L, L)] = zf
        return _
    lax.fori_loop(0, nslc // L, dz_body, None)
    for p in range(NC * NS):
        pltpu.sync_copy(den_hbm.at[p, dslice], dtmp_v)

        def dacc_body(i, _):
            sl = pl.ds(i * L, L)
            dsum_v[sl] = dsum_v[sl] + dtmp_v[sl]
            return _
        lax.fori_loop(0, nslc // L, dacc_body, None)
    pltpu.sync_copy(dsum_v, den_sh.at[dslice])
    plsc.subcore_barrier()
    pltpu.sync_copy(den_sh, den_v)

    def body(i, _):
        sl = pl.ds(i * L, L)
        dn = plsc.load_gather(den_v, [dst_v[sl]])
        ex_v[sl] = ex_v[sl] / (dn + 1e-16)
        return _
    lax.fori_loop(0, NBLK_B, body, None)

    pltpu.sync_copy(ex_v, coef_hbm.at[pl.ds(base, CHUNK_B)])


_kd = pl.kernel(
    _kd_body,
    out_type=jax.ShapeDtypeStruct((E_PAD,), jnp.float32),
    mesh=_mesh,
    compiler_params=pltpu.CompilerParams(needs_layout_passes=False),
    scratch_types=[
        pltpu.VMEM((CHUNK_B,), jnp.int32),
        pltpu.VMEM((CHUNK_B,), jnp.float32),
        pltpu.VMEM((N_PAD,), jnp.float32),
        pltpu.VMEM((N_PAD // NS,), jnp.float32),
        pltpu.VMEM((N_PAD // NS,), jnp.float32),
        pltpu.VMEM_SHARED((N_PAD,), jnp.float32),
    ],
)


CB = 32
N_CHUNK = NBLK_C // CB
NBUF = 4


def _kc_body(h0_hbm, h1_hbm, h2_hbm, h3_hbm, epk_hbm, b_hbm,
             out_hbm,
             st_v, bias_v, r0_v, r1_v, r2_v, r3_v, gsem, ssem, acc_sh):
    c = lax.axis_index("c")
    s = lax.axis_index("s")
    rows = (r0_v, r1_v, r2_v, r3_v)
    hp = (h0_hbm, h1_hbm, h2_hbm, h3_hbm)

    pltpu.sync_copy(b_hbm.at[c], bias_v)

    zf = jnp.zeros((L,), jnp.float32)
    zi = jnp.zeros((L,), jnp.int32)

    del zf
    for q in range(2):

        def rz_body(i, _):
            for g in range(4):
                r0_v[i, pl.ds(g * L, L)] = bias_v[pl.ds(q * 64 + g * L, L)]
            return _
        lax.fori_loop(0, ROW_CH, rz_body, None)
        for t in range(ROWS_T // ROW_CH):
            pltpu.sync_copy(
                r0_v, acc_sh.at[pl.ds(s * ROWS_T + t * ROW_CH, ROW_CH)])
        plsc.subcore_barrier()

        def chunk_body(ch, _):
            pltpu.sync_copy(epk_hbm.at[s, ch], st_v)

            def scale(b, j):
                cf_row = st_v.at[2, j]

                def scale_body(k, _):
                    cv = plsc.bitcast(
                        plsc.load_gather(cf_row, [zi + 2 * k]), jnp.float32)
                    cw = plsc.bitcast(
                        plsc.load_gather(cf_row, [zi + 2 * k + 1]),
                        jnp.float32)
                    for g in range(4):
                        sl = pl.ds(g * L, L)
                        rows[b][2 * k, sl] = rows[b][2 * k, sl] * cv
                        rows[b][2 * k + 1, sl] = rows[b][2 * k + 1, sl] * cw
                    return _
                lax.fori_loop(0, 64, scale_body, None)

            def issue_gather(j):
                idx = st_v.at[0, j]
                buf = rows[j % NBUF]
                sem = gsem.at[j % NBUF]
                ds = [pltpu.make_async_copy(hp[2 * cc + q].at[idx], buf, sem)
                      for cc in range(NC)]
                for cc in range(NC):
                    pl.when(c == cc)(ds[cc].start)
                return ds

            def wait_gather(ds):
                ds[0].wait()

            gd = {}
            sd = {}
            for j in range(min(2, CB)):
                gd[j] = issue_gather(j)
            for j in range(CB):
                b = j % NBUF
                wait_gather(gd[j])
                scale(b, j)
                sd[j] = pltpu.async_copy(
                    rows[b], acc_sh.at[st_v.at[1, j]], ssem.at[b], add=True)
                nj = j + 2
                if nj < CB:
                    if nj >= NBUF:
                        sd[nj - NBUF].wait()
                    gd[nj] = issue_gather(nj)
            for j in range(CB - NBUF, CB):
                sd[j].wait()
            return _
        lax.fori_loop(0, N_CHUNK, chunk_body, None)
        plsc.subcore_barrier()

        rsl = pl.ds(s * ROWS_T, ROWS_T)
        pltpu.sync_copy(acc_sh.at[rsl], out_hbm.at[2 * c + q, rsl])


_kc = pl.kernel(
    _kc_body,
    out_type=jax.ShapeDtypeStruct((2 * NC, N_PAD, 64), jnp.float32),
    mesh=_mesh,
    compiler_params=pltpu.CompilerParams(needs_layout_passes=False,
                                         use_tc_tiling_on_sc=False),
    scratch_types=[
        pltpu.VMEM((3, CB, 128), jnp.int32),
        pltpu.VMEM((HALF,), jnp.float32),
        pltpu.VMEM((ROW_CH, 64), jnp.float32),
        pltpu.VMEM((ROW_CH, 64), jnp.float32),
        pltpu.VMEM((ROW_CH, 64), jnp.float32),
        pltpu.VMEM((ROW_CH, 64), jnp.float32),
        pltpu.SemaphoreType.DMA((NBUF,)),
        pltpu.SemaphoreType.DMA((NBUF,)),
        pltpu.VMEM_SHARED((N_PAD, 64), jnp.float32),
    ],
)


def kernel(x, edge_index, edge_attr,
           W1, att_src1, att_dst1, We1, att_e1, b1,
           W2, att_src2, att_dst2, We2, att_e2, b2):
    src = edge_index[0].astype(jnp.int32)
    dst = edge_index[1].astype(jnp.int32)
    pad = E_PAD - E
    zi = jnp.zeros((pad,), jnp.int32)
    src_p = jnp.concatenate([src, zi])
    dst_p = jnp.concatenate([dst, zi])
    src5 = src_p.reshape(NS, N_CHUNK, CB, 128)
    dst5 = dst_p.reshape(NS, N_CHUNK, CB, 128)

    eat8 = jnp.zeros((8, E_PAD), jnp.float32).at[:6, :E].set(edge_attr.T)
    we1 = jnp.zeros((8, 1), jnp.float32).at[:6, 0].set(We1 @ att_e1)
    we2 = jnp.zeros((8, 1), jnp.float32).at[:6, 0].set(We2 @ att_e2)
    ae1, ae2 = _ae(eat8, we1, we2)

    def layer(h_in, W, att_s, att_d, ae, bias, relu):
        w_aug = jnp.concatenate(
            [W, (W @ att_s)[:, None], (W @ att_d)[:, None],
             jnp.zeros((W.shape[0], 126), jnp.float32)], axis=1)
        h0, h1, h2, h3, aug = _dense(h_in, w_aug, relu)
        ex, den = _kb(src_p, dst_p, ae, aug[:, 0], aug[:, 1])
        coef = _kd(dst_p, ex, den)
        cf5 = lax.bitcast_convert_type(
            coef.reshape(NS, N_CHUNK, CB, 128), jnp.int32)
        epk = jnp.stack([src5, dst5, cf5], axis=2)
        return _kc(h0, h1, h2, h3, epk, bias.reshape(NC, HALF))

    x_pad = jnp.zeros((N_PAD, IN_DIM), jnp.float32).at[:N].set(x)
    o1 = layer(x_pad, W1, att_src1, att_dst1, ae1, b1, relu=False)
    o2 = layer(o1, W2, att_src2, att_dst2, ae2, b2, relu=True)
    return jnp.concatenate([o2[0], o2[1], o2[2], o2[3]], axis=1)[:N]


# speedup vs baseline: 13.4447x; 1.0298x over previous
import functools

import jax
import jax.numpy as jnp
from jax import lax
from jax.experimental import pallas as pl
from jax.experimental.pallas import tpu as pltpu
from jax.experimental.pallas import tpu_sc as plsc

N = 10000
E = 320000
IN_DIM = 128
HID = 256
HALF = 128
NC = 2
NS = 16
L = 16

E_PAD = 327680
CHUNK_B = E_PAD // (NC * NS)
NBLK_B = CHUNK_B // L
CHUNK_C = E_PAD // NS
NBLK_C = CHUNK_C // 128
N_PAD = 10240
ROWS_T = N_PAD // NS
ROW_CH = 128

_mesh = plsc.VectorSubcoreMesh(core_axis_name="c", subcore_axis_name="s")


def _dense_body(x_ref, w_ref, ha0, ha1, ha2, ha3, aug_ref, *, relu):
    ha_ref = (ha0, ha1, ha2, ha3)
    xb = x_ref[...]
    if xb.ndim == 3:
        xb = jnp.concatenate([xb[0], xb[1], xb[2], xb[3]], axis=1)
    if relu:
        xb = jnp.maximum(xb, 0.0)
    h = jnp.dot(xb, w_ref[...], preferred_element_type=jnp.float32)
    for q in range(4):
        ha_ref[q][...] = h[:, q * 64:(q + 1) * 64]
    aug_ref[...] = h[:, HID:]


def _dense(x, w_aug, relu):
    rows = 1280
    grid = (N_PAD // rows,)
    in_dim = w_aug.shape[0]
    if x.ndim == 3:
        in_spec = pl.BlockSpec((4, rows, 64), lambda i: (0, i, 0))
    else:
        in_spec = pl.BlockSpec((rows, in_dim), lambda i: (i, 0))
    return pl.pallas_call(
        functools.partial(_dense_body, relu=relu),
        grid=grid,
        in_specs=[
            in_spec,
            pl.BlockSpec((in_dim, HID + 128), lambda i: (0, 0)),
        ],
        out_specs=[pl.BlockSpec((rows, 64), lambda i: (i, 0))] * 4 + [
            pl.BlockSpec((rows, 128), lambda i: (i, 0)),
        ],
        out_shape=[jax.ShapeDtypeStruct((N_PAD, 64), jnp.float32)] * 4 + [
            jax.ShapeDtypeStruct((N_PAD, 128), jnp.float32),
        ],
    )(x, w_aug)


def _ae_body(eat_ref, w1_ref, w2_ref, a1_ref, a2_ref, *, be):
    i = pl.program_id(0)
    blk = eat_ref[...]
    a1_ref[pl.ds(i * be, be)] = jnp.sum(blk * w1_ref[...], axis=0)
    a2_ref[pl.ds(i * be, be)] = jnp.sum(blk * w2_ref[...], axis=0)


def _ae(eat8, we1, we2):
    be = 2048
    grid = (E_PAD // be,)
    return pl.pallas_call(
        functools.partial(_ae_body, be=be),
        grid=grid,
        in_specs=[
            pl.BlockSpec((8, be), lambda i: (0, i)),
            pl.BlockSpec((8, 1), lambda i: (0, 0)),
            pl.BlockSpec((8, 1), lambda i: (0, 0)),
        ],
        out_specs=[
            pl.BlockSpec((E_PAD,), lambda i: (0,)),
            pl.BlockSpec((E_PAD,), lambda i: (0,)),
        ],
        out_shape=[
            jax.ShapeDtypeStruct((E_PAD,), jnp.float32),
            jax.ShapeDtypeStruct((E_PAD,), jnp.float32),
        ],
    )(eat8, we1, we2)


def _kb_body(src_hbm, dst_hbm, ae_hbm, asrc_hbm, adst_hbm,
             ex_hbm, den_hbm,
             src_v, dst_v, ae_v, asrc_v, adst_v, ex_v, den_v):
    wid = lax.axis_index("s") * NC + lax.axis_index("c")
    base = wid * CHUNK_B
    pltpu.sync_copy(src_hbm.at[pl.ds(base, CHUNK_B)], src_v)
    pltpu.sync_copy(dst_hbm.at[pl.ds(base, CHUNK_B)], dst_v)
    pltpu.sync_copy(ae_hbm.at[pl.ds(base, CHUNK_B)], ae_v)
    pltpu.sync_copy(asrc_hbm, asrc_v)
    pltpu.sync_copy(adst_hbm, adst_v)

    zf = jnp.zeros((L,), jnp.float32)

    def zero_body(i, _):
        den_v[pl.ds(i * L, L)] = zf
        return _
    lax.fori_loop(0, N_PAD // L, zero_body, None)

    lanes = lax.iota(jnp.int32, L)

    def body(i, _):
        sl = pl.ds(i * L, L)
        sv = src_v[sl]
        dv = dst_v[sl]
        a = plsc.load_gather(asrc_v, [sv]) + plsc.load_gather(adst_v, [dv]) \
            + ae_v[sl]
        a = jnp.maximum(a, 0.2 * a)
        ex = jnp.exp(a)
        gidx = base + i * L + lanes
        ex = jnp.where(gidx < E, ex, 0.0)
        ex_v[sl] = ex
        plsc.addupdate_scatter(den_v, [dv], ex)
        return _
    lax.fori_loop(0, NBLK_B, body, None)

    pltpu.sync_copy(ex_v, ex_hbm.at[pl.ds(base, CHUNK_B)])
    pltpu.sync_copy(den_v, den_hbm.at[wid])


_kb = pl.kernel(
    _kb_body,
    out_type=[
        jax.ShapeDtypeStruct((E_PAD,), jnp.float32),
        jax.ShapeDtypeStruct((NC * NS, N_PAD), jnp.float32),
    ],
    mesh=_mesh,
    compiler_params=pltpu.CompilerParams(needs_layout_passes=False),
    scratch_types=[
        pltpu.VMEM((CHUNK_B,), jnp.int32),
        pltpu.VMEM((CHUNK_B,), jnp.int32),
        pltpu.VMEM((CHUNK_B,), jnp.float32),
        pltpu.VMEM((N_PAD,), jnp.float32),
        pltpu.VMEM((N_PAD,), jnp.float32),
        pltpu.VMEM((CHUNK_B,), jnp.float32),
        pltpu.VMEM((N_PAD,), jnp.float32),
    ],
)


def _kd_body(dst_hbm, ex_hbm, den_hbm, coef_hbm,
             dst_v, ex_v, den_v, dtmp_v, dsum_v, den_sh):
    c = lax.axis_index("c")
    s = lax.axis_index("s")
    wid = s * NC + c
    base = wid * CHUNK_B

    pltpu.sync_copy(dst_hbm.at[pl.ds(base, CHUNK_B)], dst_v)
    pltpu.sync_copy(ex_hbm.at[pl.ds(base, CHUNK_B)], ex_v)

    nslc = N_PAD // NS
    dslice = pl.ds(s * nslc, nslc)
    zf = jnp.zeros((L,), jnp.float32)

    def dz_body(i, _):
        dsum_v[pl.ds(i * L, L)] = zf
        return _
    lax.fori_loop(0, nslc // L, dz_body, None)
    for p in range(NC * NS):
        pltpu.sync_copy(den_hbm.at[p, dslice], dtmp_v)

        def dacc_body(i, _):
            sl = pl.ds(i * L, L)
            dsum_v[sl] = dsum_v[sl] + dtmp_v[sl]
            return _
        lax.fori_loop(0, nslc // L, dacc_body, None)
    pltpu.sync_copy(dsum_v, den_sh.at[dslice])
    plsc.subcore_barrier()
    pltpu.sync_copy(den_sh, den_v)

    def body(i, _):
        sl = pl.ds(i * L, L)
        dn = plsc.load_gather(den_v, [dst_v[sl]])
        ex_v[sl] = ex_v[sl] / (dn + 1e-16)
        return _
    lax.fori_loop(0, NBLK_B, body, None)

    pltpu.sync_copy(ex_v, coef_hbm.at[pl.ds(base, CHUNK_B)])


_kd = pl.kernel(
    _kd_body,
    out_type=jax.ShapeDtypeStruct((E_PAD,), jnp.float32),
    mesh=_mesh,
    compiler_params=pltpu.CompilerParams(needs_layout_passes=False),
    scratch_types=[
        pltpu.VMEM((CHUNK_B,), jnp.int32),
        pltpu.VMEM((CHUNK_B,), jnp.float32),
        pltpu.VMEM((N_PAD,), jnp.float32),
        pltpu.VMEM((N_PAD // NS,), jnp.float32),
        pltpu.VMEM((N_PAD // NS,), jnp.float32),
        pltpu.VMEM_SHARED((N_PAD,), jnp.float32),
    ],
)


CB = 32
N_CHUNK = NBLK_C // CB
NBUF = 4


def _kc_body(h0_hbm, h1_hbm, h2_hbm, h3_hbm, epk_hbm, b_hbm,
             out_hbm,
             st_v, bias_v, r0_v, r1_v, r2_v, r3_v, gsem, ssem, acc_sh):
    c = lax.axis_index("c")
    s = lax.axis_index("s")
    rows = (r0_v, r1_v, r2_v, r3_v)
    hp = (h0_hbm, h1_hbm, h2_hbm, h3_hbm)

    pltpu.sync_copy(b_hbm.at[c], bias_v)

    zf = jnp.zeros((L,), jnp.float32)
    zi = jnp.zeros((L,), jnp.int32)

    del zf
    for q in range(2):

        def rz_body(i, _):
            for g in range(4):
                r0_v[i, pl.ds(g * L, L)] = bias_v[pl.ds(q * 64 + g * L, L)]
            return _
        lax.fori_loop(0, ROW_CH, rz_body, None)
        for t in range(ROWS_T // ROW_CH):
            pltpu.sync_copy(
                r0_v, acc_sh.at[pl.ds(s * ROWS_T + t * ROW_CH, ROW_CH)])
        plsc.subcore_barrier()

        def chunk_body(ch, _):
            pltpu.sync_copy(epk_hbm.at[s, ch], st_v)

            def scale(b, j):
                cf_row = st_v.at[2, j]

                def scale_body(k, _):
                    cv = plsc.bitcast(
                        plsc.load_gather(cf_row, [zi + 2 * k]), jnp.float32)
                    cw = plsc.bitcast(
                        plsc.load_gather(cf_row, [zi + 2 * k + 1]),
                        jnp.float32)
                    for g in range(4):
                        sl = pl.ds(g * L, L)
                        rows[b][2 * k, sl] = rows[b][2 * k, sl] * cv
                        rows[b][2 * k + 1, sl] = rows[b][2 * k + 1, sl] * cw
                    return _
                lax.fori_loop(0, 64, scale_body, None)

            def issue_gather(j):
                idx = st_v.at[0, j]
                buf = rows[j % NBUF]
                sem = gsem.at[j % NBUF]
                ds = [pltpu.make_async_copy(hp[2 * cc + q].at[idx], buf, sem)
                      for cc in range(NC)]
                for cc in range(NC):
                    pl.when(c == cc)(ds[cc].start)
                return ds

            def wait_gather(ds):
                ds[0].wait()

            gd = {}
            sd = {}
            for j in range(min(3, CB)):
                gd[j] = issue_gather(j)
            for j in range(CB):
                b = j % NBUF
                wait_gather(gd[j])
                scale(b, j)
                sd[j] = pltpu.async_copy(
                    rows[b], acc_sh.at[st_v.at[1, j]], ssem.at[b], add=True)
                nj = j + 3
                if nj < CB:
                    if nj >= NBUF:
                        sd[nj - NBUF].wait()
                    gd[nj] = issue_gather(nj)
            for j in range(CB - NBUF, CB):
                sd[j].wait()
            return _
        lax.fori_loop(0, N_CHUNK, chunk_body, None)
        plsc.subcore_barrier()

        rsl = pl.ds(s * ROWS_T, ROWS_T)
        pltpu.sync_copy(acc_sh.at[rsl], out_hbm.at[2 * c + q, rsl])


_kc = pl.kernel(
    _kc_body,
    out_type=jax.ShapeDtypeStruct((2 * NC, N_PAD, 64), jnp.float32),
    mesh=_mesh,
    compiler_params=pltpu.CompilerParams(needs_layout_passes=False,
                                         use_tc_tiling_on_sc=False),
    scratch_types=[
        pltpu.VMEM((3, CB, 128), jnp.int32),
        pltpu.VMEM((HALF,), jnp.float32),
        pltpu.VMEM((ROW_CH, 64), jnp.float32),
        pltpu.VMEM((ROW_CH, 64), jnp.float32),
        pltpu.VMEM((ROW_CH, 64), jnp.float32),
        pltpu.VMEM((ROW_CH, 64), jnp.float32),
        pltpu.SemaphoreType.DMA((NBUF,)),
        pltpu.SemaphoreType.DMA((NBUF,)),
        pltpu.VMEM_SHARED((N_PAD, 64), jnp.float32),
    ],
)


def kernel(x, edge_index, edge_attr,
           W1, att_src1, att_dst1, We1, att_e1, b1,
           W2, att_src2, att_dst2, We2, att_e2, b2):
    src = edge_index[0].astype(jnp.int32)
    dst = edge_index[1].astype(jnp.int32)
    pad = E_PAD - E
    zi = jnp.zeros((pad,), jnp.int32)
    src_p = jnp.concatenate([src, zi])
    dst_p = jnp.concatenate([dst, zi])
    src5 = src_p.reshape(NS, N_CHUNK, CB, 128)
    dst5 = dst_p.reshape(NS, N_CHUNK, CB, 128)

    eat8 = jnp.zeros((8, E_PAD), jnp.float32).at[:6, :E].set(edge_attr.T)
    we1 = jnp.zeros((8, 1), jnp.float32).at[:6, 0].set(We1 @ att_e1)
    we2 = jnp.zeros((8, 1), jnp.float32).at[:6, 0].set(We2 @ att_e2)
    ae1, ae2 = _ae(eat8, we1, we2)

    def layer(h_in, W, att_s, att_d, ae, bias, relu):
        w_aug = jnp.concatenate(
            [W, (W @ att_s)[:, None], (W @ att_d)[:, None],
             jnp.zeros((W.shape[0], 126), jnp.float32)], axis=1)
        h0, h1, h2, h3, aug = _dense(h_in, w_aug, relu)
        ex, den = _kb(src_p, dst_p, ae, aug[:, 0], aug[:, 1])
        coef = _kd(dst_p, ex, den)
        cf5 = lax.bitcast_convert_type(
            coef.reshape(NS, N_CHUNK, CB, 128), jnp.int32)
        epk = jnp.stack([src5, dst5, cf5], axis=2)
        return _kc(h0, h1, h2, h3, epk, bias.reshape(NC, HALF))

    x_pad = jnp.zeros((N_PAD, IN_DIM), jnp.float32).at[:N].set(x)
    o1 = layer(x_pad, W1, att_src1, att_dst1, ae1, b1, relu=False)
    o2 = layer(o1, W2, att_src2, att_dst2, ae2, b2, relu=True)
    return jnp.concatenate([o2[0], o2[1], o2[2], o2[3]], axis=1)[:N]
